# baseline (speedup 1.0000x reference)
import functools

import jax
import jax.numpy as jnp
from jax.experimental import pallas as pl
from jax.experimental.pallas import tpu as pltpu

P_PAD = 24


def _dot(x, y):
    return jax.lax.dot_general(
        x.astype(jnp.bfloat16), y.astype(jnp.bfloat16),
        (((1,), (0,)), ((), ())), preferred_element_type=jnp.float32)


def _dot_tl(x, y):
    return jax.lax.dot_general(
        x.astype(jnp.bfloat16), y.astype(jnp.bfloat16),
        (((0,), (0,)), ((), ())), preferred_element_type=jnp.float32)


def _tc_body(feat_ref, scal_ref, W1p_ref, b1_ref, W2_ref, b2_ref,
             W3a_ref, W3b_ref, b3_ref, W4_ref, b4_ref,
             slW1_ref, slb1_ref, slW2_ref, slb2_ref,
             Wemb_ref, unk_ref, out_ref, *, B):
    BP = B * P_PAD
    a = feat_ref[...]
    vmrep = _dot_tl(a[6:7, :], jnp.ones((1, 256), jnp.float32))
    h1 = jnp.maximum(_dot_tl(a, W1p_ref[...]) + b1_ref[...], 0.0)
    h = (_dot(h1, W2_ref[...]) + b2_ref[...]) * vmrep
    pooled = jnp.max(h.reshape(B, P_PAD, 256), axis=1)
    g2 = _dot(pooled, W3b_ref[...]) + b3_ref[...]
    t1 = _dot(h, W3a_ref[...])
    h3 = jnp.maximum(t1.reshape(B, P_PAD, 256) + g2[:, None, :], 0.0)
    h3 = h3.reshape(BP, 256)
    h4 = (_dot(h3, W4_ref[...]) + b4_ref[...]) * vmrep[:, :128]
    x_poly = jnp.max(h4.reshape(B, P_PAD, 128), axis=1)

    s = scal_ref[...]
    sl1 = jnp.maximum(_dot(s[:, 0:1], slW1_ref[...]) + slb1_ref[...], 0.0)
    sl = _dot(sl1, slW2_ref[...]) + slb2_ref[...]
    hs = _dot(s[:, 1:2], jnp.ones((1, 128), jnp.float32))
    emb = _dot(s, Wemb_ref[...]) + unk_ref[...]
    out_ref[...] = x_poly + emb + hs * sl


def kernel(polygon_center, polygon_type, polygon_on_route, polygon_tl_status,
           polygon_has_speed_limit, polygon_speed_limit, point_position,
           point_vector, point_orientation, valid_mask,
           pe_W1, pe_b1, pe_W2, pe_b2, pe_W3, pe_b3, pe_W4, pe_b4,
           sl_W1, sl_b1, sl_W2, sl_b2, type_emb, on_route_emb, tl_emb,
           unknown_speed_emb):
    bs, M, P = point_orientation.shape[0], point_orientation.shape[1], point_orientation.shape[3]
    N = bs * M
    B = 128

    pt_pos = point_position[:, :, 0] - polygon_center[..., None, :2]
    vec = point_vector[:, :, 0]
    ori = point_orientation[:, :, 0]
    vmf = valid_mask.astype(jnp.float32)
    pad = lambda x: jnp.pad(x, ((0, 0), (0, 0), (0, P_PAD - P)))
    feat = jnp.stack([pad(pt_pos[..., 0]), pad(pt_pos[..., 1]),
                      pad(vec[..., 0]), pad(vec[..., 1]),
                      pad(jnp.cos(ori)), pad(jnp.sin(ori)), pad(vmf),
                      jnp.zeros((bs, M, P_PAD), jnp.float32)], axis=0)
    feat = feat.reshape(8, N * P_PAD)

    def onehot(idx, k):
        return (idx[..., None] == jnp.arange(k)).astype(jnp.float32)

    scal = jnp.concatenate(
        [polygon_speed_limit[..., None],
         polygon_has_speed_limit.astype(jnp.float32)[..., None],
         onehot(polygon_type, 3), onehot(polygon_on_route, 2),
         onehot(polygon_tl_status, 4),
         jnp.zeros((bs, M, 5), jnp.float32)], axis=-1)
    scal = scal.reshape(N, 16)

    W1p = jnp.zeros((8, 128), jnp.float32).at[:6].set(pe_W1)
    W3a, W3b = pe_W3[:256], pe_W3[256:]
    Wemb = jnp.concatenate(
        [jnp.zeros((1, 128), jnp.float32), -unknown_speed_emb,
         type_emb, on_route_emb, tl_emb,
         jnp.zeros((5, 128), jnp.float32)], axis=0)
    row = lambda b: b.reshape(1, -1)

    grid = N // B
    const = lambda shape: pl.BlockSpec(shape, lambda i: (0, 0))
    out = pl.pallas_call(
        functools.partial(_tc_body, B=B),
        grid=(grid,),
        in_specs=[
            pl.BlockSpec((8, B * P_PAD), lambda i: (0, i)),
            pl.BlockSpec((B, 16), lambda i: (i, 0)),
            const((8, 128)), const((1, 128)),
            const((128, 256)), const((1, 256)),
            const((256, 256)), const((256, 256)), const((1, 256)),
            const((256, 128)), const((1, 128)),
            const((1, 128)), const((1, 128)),
            const((128, 128)), const((1, 128)),
            const((16, 128)), const((1, 128)),
        ],
        out_specs=pl.BlockSpec((B, 128), lambda i: (i, 0)),
        out_shape=jax.ShapeDtypeStruct((N, 128), jnp.float32),
    )(feat, scal, W1p, row(pe_b1), pe_W2, row(pe_b2),
      W3a, W3b, row(pe_b3), pe_W4, row(pe_b4),
      sl_W1, row(sl_b1), sl_W2, row(sl_b2),
      Wemb, unknown_speed_emb)
    return out.reshape(bs, M, 128)


# speedup vs baseline: 1.0570x; 1.0570x over previous
import functools

import jax
import jax.numpy as jnp
from jax.experimental import pallas as pl
from jax.experimental.pallas import tpu as pltpu

P_PAD = 24


def _dot(x, y):
    return jax.lax.dot_general(
        x.astype(jnp.bfloat16), y.astype(jnp.bfloat16),
        (((1,), (0,)), ((), ())), preferred_element_type=jnp.float32)


def _tc_body(feat_ref, scal_ref, W1p_ref, b1_ref, W2_ref, b2_ref,
             W3a_ref, W3b_ref, b3_ref, W4_ref, b4_ref,
             slW1_ref, slb1_ref, slW2_ref, slb2_ref,
             Wemb_ref, unk_ref, out_ref, *, B):
    BP = B * P_PAD
    a = feat_ref[...]
    vmrep = _dot(a[:, 6:7], jnp.ones((1, 256), jnp.float32))
    h1 = jnp.maximum(_dot(a, W1p_ref[...]) + b1_ref[...], 0.0)
    h = (_dot(h1, W2_ref[...]) + b2_ref[...]) * vmrep
    pooled = jnp.max(h.reshape(B, P_PAD, 256), axis=1)
    g2 = _dot(pooled, W3b_ref[...]) + b3_ref[...]
    t1 = _dot(h, W3a_ref[...])
    h3 = jnp.maximum(t1.reshape(B, P_PAD, 256) + g2[:, None, :], 0.0)
    h3 = h3.reshape(BP, 256)
    h4 = (_dot(h3, W4_ref[...]) + b4_ref[...]) * vmrep[:, :128]
    x_poly = jnp.max(h4.reshape(B, P_PAD, 128), axis=1)

    s = scal_ref[...]
    sl1 = jnp.maximum(_dot(s[:, 0:1], slW1_ref[...]) + slb1_ref[...], 0.0)
    sl = _dot(sl1, slW2_ref[...]) + slb2_ref[...]
    hs = _dot(s[:, 1:2], jnp.ones((1, 128), jnp.float32))
    emb = _dot(s, Wemb_ref[...]) + unk_ref[...]
    out_ref[...] = x_poly + emb + hs * sl


def kernel(polygon_center, polygon_type, polygon_on_route, polygon_tl_status,
           polygon_has_speed_limit, polygon_speed_limit, point_position,
           point_vector, point_orientation, valid_mask,
           pe_W1, pe_b1, pe_W2, pe_b2, pe_W3, pe_b3, pe_W4, pe_b4,
           sl_W1, sl_b1, sl_W2, sl_b2, type_emb, on_route_emb, tl_emb,
           unknown_speed_emb):
    bs, M, P = point_orientation.shape[0], point_orientation.shape[1], point_orientation.shape[3]
    N = bs * M
    B = 128

    bf = jnp.bfloat16
    pt_pos = (point_position[:, :, 0]
              - polygon_center[..., None, :2]).astype(bf)
    vec = point_vector[:, :, 0].astype(bf)
    ori = point_orientation[:, :, 0]
    trig = jnp.stack([jnp.cos(ori), jnp.sin(ori)], axis=-1).astype(bf)
    vmz = jnp.stack([valid_mask.astype(bf),
                     jnp.zeros(valid_mask.shape, bf)], axis=-1)
    rows = jnp.concatenate([pt_pos, vec, trig, vmz], axis=-1)
    pad = P_PAD - P
    feat = jnp.concatenate(
        [rows, jnp.zeros((bs, M, pad, 8), bf)], axis=2).reshape(N * P_PAD, 8)

    def onehot(idx, k):
        return (idx[..., None] == jnp.arange(k)).astype(bf)

    scal = jnp.concatenate(
        [polygon_speed_limit[..., None].astype(bf),
         polygon_has_speed_limit[..., None].astype(bf),
         onehot(polygon_type, 3), onehot(polygon_on_route, 2),
         onehot(polygon_tl_status, 4),
         jnp.zeros((bs, M, 5), bf)], axis=-1)
    scal = scal.reshape(N, 16)

    W1p = jnp.zeros((8, 128), jnp.float32).at[:6].set(pe_W1)
    W3a, W3b = pe_W3[:256], pe_W3[256:]
    Wemb = jnp.concatenate(
        [jnp.zeros((1, 128), jnp.float32), -unknown_speed_emb,
         type_emb, on_route_emb, tl_emb,
         jnp.zeros((5, 128), jnp.float32)], axis=0)
    row = lambda b: b.reshape(1, -1)

    grid = N // B
    const = lambda shape: pl.BlockSpec(shape, lambda i: (0, 0))
    out = pl.pallas_call(
        functools.partial(_tc_body, B=B),
        grid=(grid,),
        in_specs=[
            pl.BlockSpec((B * P_PAD, 8), lambda i: (i, 0)),
            pl.BlockSpec((B, 16), lambda i: (i, 0)),
            const((8, 128)), const((1, 128)),
            const((128, 256)), const((1, 256)),
            const((256, 256)), const((256, 256)), const((1, 256)),
            const((256, 128)), const((1, 128)),
            const((1, 128)), const((1, 128)),
            const((128, 128)), const((1, 128)),
            const((16, 128)), const((1, 128)),
        ],
        out_specs=pl.BlockSpec((B, 128), lambda i: (i, 0)),
        out_shape=jax.ShapeDtypeStruct((N, 128), jnp.float32),
    )(feat, scal, W1p, row(pe_b1), pe_W2, row(pe_b2),
      W3a, W3b, row(pe_b3), pe_W4, row(pe_b4),
      sl_W1, row(sl_b1), sl_W2, row(sl_b2),
      Wemb, unknown_speed_emb)
    return out.reshape(bs, M, 128)


# speedup vs baseline: 1.1329x; 1.0718x over previous
import functools

import jax
import jax.numpy as jnp
from jax.experimental import pallas as pl
from jax.experimental.pallas import tpu as pltpu

P_PAD = 24


def _dot(x, y):
    return jax.lax.dot_general(
        x.astype(jnp.bfloat16), y.astype(jnp.bfloat16),
        (((1,), (0,)), ((), ())), preferred_element_type=jnp.float32)


def _tc_body(feat_ref, scal_ref, W1p_ref, b1_ref, W2_ref, b2_ref,
             W3a_ref, W3b_ref, b3_ref, W4_ref, b4_ref,
             slW1_ref, slb1_ref, slW2_ref, slb2_ref,
             Wemb_ref, unk_ref, out_ref, *, B):
    BP = B * P_PAD
    a = feat_ref[...]
    vmrep = _dot(a[:, 6:7], jnp.ones((1, 256), jnp.float32))
    h1 = jnp.maximum(_dot(a, W1p_ref[...]) + b1_ref[...], 0.0)
    h = (_dot(h1, W2_ref[...]) + b2_ref[...]) * vmrep
    pooled = jnp.max(h.reshape(B, P_PAD, 256), axis=1)
    g2 = _dot(pooled, W3b_ref[...]) + b3_ref[...]
    t1 = _dot(h, W3a_ref[...])
    h3 = jnp.maximum(t1.reshape(B, P_PAD, 256) + g2[:, None, :], 0.0)
    h3 = h3.reshape(BP, 256)
    h4 = (_dot(h3, W4_ref[...]) + b4_ref[...]) * vmrep[:, :128]
    x_poly = jnp.max(h4.reshape(B, P_PAD, 128), axis=1)

    s = scal_ref[...]
    sl1 = jnp.maximum(_dot(s[:, 0:1], slW1_ref[...]) + slb1_ref[...], 0.0)
    sl = _dot(sl1, slW2_ref[...]) + slb2_ref[...]
    hs = _dot(s[:, 1:2], jnp.ones((1, 128), jnp.float32))
    emb = _dot(s, Wemb_ref[...]) + unk_ref[...]
    out_ref[...] = x_poly + emb + hs * sl


def kernel(polygon_center, polygon_type, polygon_on_route, polygon_tl_status,
           polygon_has_speed_limit, polygon_speed_limit, point_position,
           point_vector, point_orientation, valid_mask,
           pe_W1, pe_b1, pe_W2, pe_b2, pe_W3, pe_b3, pe_W4, pe_b4,
           sl_W1, sl_b1, sl_W2, sl_b2, type_emb, on_route_emb, tl_emb,
           unknown_speed_emb):
    bs, M, P = point_orientation.shape[0], point_orientation.shape[1], point_orientation.shape[3]
    N = bs * M
    B = 256

    bf = jnp.bfloat16
    pt_pos = (point_position[:, :, 0]
              - polygon_center[..., None, :2]).astype(bf)
    vec = point_vector[:, :, 0].astype(bf)
    ori = point_orientation[:, :, 0]
    trig = jnp.stack([jnp.cos(ori), jnp.sin(ori)], axis=-1).astype(bf)
    vmz = jnp.stack([valid_mask.astype(bf),
                     jnp.zeros(valid_mask.shape, bf)], axis=-1)
    rows = jnp.concatenate([pt_pos, vec, trig, vmz], axis=-1)
    pad = P_PAD - P
    feat = jnp.concatenate(
        [rows, jnp.zeros((bs, M, pad, 8), bf)], axis=2).reshape(N * P_PAD, 8)

    def onehot(idx, k):
        return (idx[..., None] == jnp.arange(k)).astype(bf)

    scal = jnp.concatenate(
        [polygon_speed_limit[..., None].astype(bf),
         polygon_has_speed_limit[..., None].astype(bf),
         onehot(polygon_type, 3), onehot(polygon_on_route, 2),
         onehot(polygon_tl_status, 4),
         jnp.zeros((bs, M, 5), bf)], axis=-1)
    scal = scal.reshape(N, 16)

    W1p = jnp.zeros((8, 128), jnp.float32).at[:6].set(pe_W1)
    W3a, W3b = pe_W3[:256], pe_W3[256:]
    Wemb = jnp.concatenate(
        [jnp.zeros((1, 128), jnp.float32), -unknown_speed_emb,
         type_emb, on_route_emb, tl_emb,
         jnp.zeros((5, 128), jnp.float32)], axis=0)
    row = lambda b: b.reshape(1, -1)

    grid = N // B
    const = lambda shape: pl.BlockSpec(shape, lambda i: (0, 0))
    out = pl.pallas_call(
        functools.partial(_tc_body, B=B),
        grid=(grid,),
        in_specs=[
            pl.BlockSpec((B * P_PAD, 8), lambda i: (i, 0)),
            pl.BlockSpec((B, 16), lambda i: (i, 0)),
            const((8, 128)), const((1, 128)),
            const((128, 256)), const((1, 256)),
            const((256, 256)), const((256, 256)), const((1, 256)),
            const((256, 128)), const((1, 128)),
            const((1, 128)), const((1, 128)),
            const((128, 128)), const((1, 128)),
            const((16, 128)), const((1, 128)),
        ],
        out_specs=pl.BlockSpec((B, 128), lambda i: (i, 0)),
        out_shape=jax.ShapeDtypeStruct((N, 128), jnp.float32),
    )(feat, scal, W1p, row(pe_b1), pe_W2, row(pe_b2),
      W3a, W3b, row(pe_b3), pe_W4, row(pe_b4),
      sl_W1, row(sl_b1), sl_W2, row(sl_b2),
      Wemb, unknown_speed_emb)
    return out.reshape(bs, M, 128)


# speedup vs baseline: 1.1762x; 1.0382x over previous
import functools

import jax
import jax.numpy as jnp
from jax.experimental import pallas as pl
from jax.experimental.pallas import tpu as pltpu

P_PAD = 24


def _dot(x, y):
    return jax.lax.dot_general(
        x.astype(jnp.bfloat16), y.astype(jnp.bfloat16),
        (((1,), (0,)), ((), ())), preferred_element_type=jnp.float32)


def _tc_body(feat_ref, scal_ref, W1p_ref, b1_ref, W2_ref, b2_ref,
             W3a_ref, W3b_ref, b3_ref, W4_ref, b4_ref,
             slW1_ref, slb1_ref, slW2_ref, slb2_ref,
             Wemb_ref, unk_ref, out_ref, *, B):
    BP = B * P_PAD
    a = feat_ref[...]
    vmrep = _dot(a[:, 6:7], jnp.ones((1, 256), jnp.float32))
    h1 = jnp.maximum(_dot(a, W1p_ref[...]) + b1_ref[...], 0.0)
    h = (_dot(h1, W2_ref[...]) + b2_ref[...]) * vmrep
    pooled = jnp.max(h.reshape(B, P_PAD, 256), axis=1)
    g2 = _dot(pooled, W3b_ref[...]) + b3_ref[...]
    t1 = _dot(h, W3a_ref[...])
    h3 = jnp.maximum(t1.reshape(B, P_PAD, 256) + g2[:, None, :], 0.0)
    h3 = h3.reshape(BP, 256)
    h4 = (_dot(h3, W4_ref[...]) + b4_ref[...]) * vmrep[:, :128]
    x_poly = jnp.max(h4.reshape(B, P_PAD, 128), axis=1)

    s = scal_ref[...]
    sl1 = jnp.maximum(_dot(s[:, 0:1], slW1_ref[...]) + slb1_ref[...], 0.0)
    sl = _dot(sl1, slW2_ref[...]) + slb2_ref[...]
    hs = _dot(s[:, 1:2], jnp.ones((1, 128), jnp.float32))
    emb = _dot(s, Wemb_ref[...]) + unk_ref[...]
    out_ref[...] = x_poly + emb + hs * sl


def kernel(polygon_center, polygon_type, polygon_on_route, polygon_tl_status,
           polygon_has_speed_limit, polygon_speed_limit, point_position,
           point_vector, point_orientation, valid_mask,
           pe_W1, pe_b1, pe_W2, pe_b2, pe_W3, pe_b3, pe_W4, pe_b4,
           sl_W1, sl_b1, sl_W2, sl_b2, type_emb, on_route_emb, tl_emb,
           unknown_speed_emb):
    bs, M, P = point_orientation.shape[0], point_orientation.shape[1], point_orientation.shape[3]
    N = bs * M
    B = 512

    bf = jnp.bfloat16
    pt_pos = (point_position[:, :, 0]
              - polygon_center[..., None, :2]).astype(bf)
    vec = point_vector[:, :, 0].astype(bf)
    ori = point_orientation[:, :, 0]
    trig = jnp.stack([jnp.cos(ori), jnp.sin(ori)], axis=-1).astype(bf)
    vmz = jnp.stack([valid_mask.astype(bf),
                     jnp.zeros(valid_mask.shape, bf)], axis=-1)
    rows = jnp.concatenate([pt_pos, vec, trig, vmz], axis=-1)
    pad = P_PAD - P
    feat = jnp.concatenate(
        [rows, jnp.zeros((bs, M, pad, 8), bf)], axis=2).reshape(N * P_PAD, 8)

    def onehot(idx, k):
        return (idx[..., None] == jnp.arange(k)).astype(bf)

    scal = jnp.concatenate(
        [polygon_speed_limit[..., None].astype(bf),
         polygon_has_speed_limit[..., None].astype(bf),
         onehot(polygon_type, 3), onehot(polygon_on_route, 2),
         onehot(polygon_tl_status, 4),
         jnp.zeros((bs, M, 5), bf)], axis=-1)
    scal = scal.reshape(N, 16)

    W1p = jnp.zeros((8, 128), jnp.float32).at[:6].set(pe_W1)
    W3a, W3b = pe_W3[:256], pe_W3[256:]
    Wemb = jnp.concatenate(
        [jnp.zeros((1, 128), jnp.float32), -unknown_speed_emb,
         type_emb, on_route_emb, tl_emb,
         jnp.zeros((5, 128), jnp.float32)], axis=0)
    row = lambda b: b.reshape(1, -1)

    grid = N // B
    const = lambda shape: pl.BlockSpec(shape, lambda i: (0, 0))
    out = pl.pallas_call(
        functools.partial(_tc_body, B=B),
        grid=(grid,),
        in_specs=[
            pl.BlockSpec((B * P_PAD, 8), lambda i: (i, 0)),
            pl.BlockSpec((B, 16), lambda i: (i, 0)),
            const((8, 128)), const((1, 128)),
            const((128, 256)), const((1, 256)),
            const((256, 256)), const((256, 256)), const((1, 256)),
            const((256, 128)), const((1, 128)),
            const((1, 128)), const((1, 128)),
            const((128, 128)), const((1, 128)),
            const((16, 128)), const((1, 128)),
        ],
        out_specs=pl.BlockSpec((B, 128), lambda i: (i, 0)),
        out_shape=jax.ShapeDtypeStruct((N, 128), jnp.float32),
    )(feat, scal, W1p, row(pe_b1), pe_W2, row(pe_b2),
      W3a, W3b, row(pe_b3), pe_W4, row(pe_b4),
      sl_W1, row(sl_b1), sl_W2, row(sl_b2),
      Wemb, unknown_speed_emb)
    return out.reshape(bs, M, 128)
